# Initial kernel scaffold; baseline (speedup 1.0000x reference)
import jax
import jax.numpy as jnp
from jax.experimental import pallas as pl


def kernel(xyz, params):
    raise NotImplementedError("write your pallas kernel here")


# speedup vs baseline: 5.0383x; 5.0383x over previous
import functools

import jax
import jax.numpy as jnp
from jax import lax
from jax.experimental import pallas as pl
from jax.experimental.pallas import tpu as pltpu
from jax.experimental.pallas import tpu_sc as plsc

F32 = jnp.float32
NEG = -1e30


def _fold(layer):
    s = layer['gamma'] / jnp.sqrt(1.0 + 1e-5)
    w = layer['w'] * s[:, None]
    b = layer['b'] * s + layer['beta']
    return w.T, b[None, :]


def _fps_body(x_ref, o_ref, *, N, S):
    xp = x_ref[0]
    lane = lax.broadcasted_iota(jnp.int32, (1, N), 1)
    laneS = lax.broadcasted_iota(jnp.int32, (8, S), 1)

    def step(i, carry):
        dist, far, cent = carry
        sel = (lane == far).astype(F32)
        vfar = jnp.sum(xp * sel, axis=1, keepdims=True)
        cent = jnp.where(laneS == i, vfar, cent)
        d = jnp.sum((xp - vfar) ** 2, axis=0, keepdims=True)
        dist = jnp.minimum(dist, d)
        m = jnp.max(dist)
        far = jnp.min(jnp.where(dist == m, lane, N)).astype(jnp.int32)
        return dist, far, cent

    init = (jnp.full((1, N), 1e10, F32), jnp.int32(0), jnp.zeros((8, S), F32))
    _, _, cent = lax.fori_loop(0, S, step, init)
    o_ref[0] = cent


def _fps(xyzp, S):
    B, _, N = xyzp.shape
    return pl.pallas_call(
        functools.partial(_fps_body, N=N, S=S),
        grid=(B,),
        in_specs=[pl.BlockSpec((1, 8, N), lambda b: (b, 0, 0))],
        out_specs=pl.BlockSpec((1, 8, S), lambda b: (b, 0, 0)),
        out_shape=jax.ShapeDtypeStruct((B, 8, S), F32),
    )(xyzp)


def _probe(rank, pos, nspans):
    hi = pos // 128
    lo = pos - hi * 128
    acc = jnp.zeros(pos.shape, F32)
    for j in range(nspans):
        g = jnp.take_along_axis(rank[:, j * 128:(j + 1) * 128], lo, axis=1)
        acc = jnp.where(hi == j, g, acc)
    return acc


def _ballq_body(x_ref, c_ref, gi_ref, cnt_ref, *, N, K, r2, Sq):
    xp = x_ref[0]
    ce = c_ref[0]
    G = lax.dot_general(ce.astype(jnp.bfloat16), xp.astype(jnp.bfloat16),
                        (((1,), (0,)), ((), ())),
                        preferred_element_type=F32)
    sn = jnp.sum(ce * ce, axis=1, keepdims=True)
    sx = jnp.sum(xp * xp, axis=0, keepdims=True)
    D = sn + sx - 2.0 * G
    mask = (D <= r2).astype(F32)

    lane = lax.broadcasted_iota(jnp.int32, (Sq, N), 1)
    r = mask
    sh = 1
    while sh < N:
        r = r + jnp.where(lane >= sh, jnp.roll(r, sh, axis=1), 0.0)
        sh *= 2
    cnt = r[:, N - 1:N]
    cnt_ref[0] = jnp.broadcast_to(cnt, (Sq, 8))

    kk = (lax.broadcasted_iota(jnp.int32, (Sq, K), 1) + 1).astype(F32)
    lo = jnp.full((Sq, K), -1, jnp.int32)
    b = 1
    while b < N:
        b *= 2
    while b >= 1:
        nxt = lo + b
        v = _probe(r, jnp.minimum(nxt, N - 1), N // 128)
        ok = (nxt <= N - 1) & (v < kk)
        lo = jnp.where(ok, nxt, lo)
        b //= 2
    g = jnp.minimum(lo + 1, N - 1)
    gi_ref[0] = g + pl.program_id(0) * N


def _ballq(xyzp, centrows, K, r2, Sq):
    B, _, N = xyzp.shape
    S = centrows.shape[1]
    return pl.pallas_call(
        functools.partial(_ballq_body, N=N, K=K, r2=r2, Sq=Sq),
        grid=(B, S // Sq),
        in_specs=[
            pl.BlockSpec((1, 8, N), lambda b, j: (b, 0, 0)),
            pl.BlockSpec((1, Sq, 8), lambda b, j: (b, j, 0)),
        ],
        out_specs=[
            pl.BlockSpec((1, Sq, K), lambda b, j: (b, j, 0)),
            pl.BlockSpec((1, Sq, 8), lambda b, j: (b, j, 0)),
        ],
        out_shape=[
            jax.ShapeDtypeStruct((B, S, K), jnp.int32),
            jax.ShapeDtypeStruct((B, S, 8), F32),
        ],
    )(xyzp, centrows)


def _sc_gather(table, idx):
    R, Dc = table.shape
    (Bt,) = idx.shape
    info = plsc.get_sparse_core_info()
    NW = info.num_cores * info.num_subcores
    b_per_w = Bt // NW
    limit = 120000 // (Dc + 1)
    chunk = b_per_w
    while chunk > limit:
        chunk //= 2
    nchunks = b_per_w // chunk
    mesh = plsc.VectorSubcoreMesh(core_axis_name="c", subcore_axis_name="s")

    @functools.partial(
        pl.kernel, mesh=mesh,
        out_type=jax.ShapeDtypeStruct((Bt, Dc), F32),
        scratch_types=[
            pltpu.VMEM((chunk,), jnp.int32),
            pltpu.VMEM((chunk, Dc), F32),
            pltpu.SemaphoreType.DMA,
        ],
    )
    def k(table_hbm, idx_hbm, out_hbm, idx_v, rows_v, sem):
        wid = lax.axis_index("s") * info.num_cores + lax.axis_index("c")
        base = wid * b_per_w

        def body(i, c):
            off = base + i * chunk
            pltpu.sync_copy(idx_hbm.at[pl.ds(off, chunk)], idx_v)
            pltpu.async_copy(table_hbm.at[idx_v], rows_v, sem).wait()
            pltpu.sync_copy(rows_v, out_hbm.at[pl.ds(off, chunk)])
            return c

        lax.fori_loop(0, nchunks, body, 0)

    return k(table, idx)


def _dense_body(x_ref, w_ref, b_ref, o_ref):
    o_ref[0] = jnp.dot(x_ref[0], w_ref[...],
                       preferred_element_type=F32, precision=jax.lax.Precision.HIGHEST) + b_ref[...]


def _dense(x, w, b):
    B, N, Ci = x.shape
    Co = w.shape[1]
    return pl.pallas_call(
        _dense_body,
        grid=(B,),
        in_specs=[
            pl.BlockSpec((1, N, Ci), lambda bb: (bb, 0, 0)),
            pl.BlockSpec((Ci, Co), lambda bb: (0, 0)),
            pl.BlockSpec((1, Co), lambda bb: (0, 0)),
        ],
        out_specs=pl.BlockSpec((1, N, Co), lambda bb: (bb, 0, 0)),
        out_shape=jax.ShapeDtypeStruct((B, N, Co), F32),
    )(x, w, b)


def _mlppool_body(g_ref, c_ref, cnt_ref, wx_ref, w2_ref, b2_ref, w3_ref,
                  b3_ref, o_ref, *, Sb, K, c3):
    gx = g_ref[0, 0]
    ce = c_ref[0]
    corr = lax.dot_general(ce, wx_ref[...], (((1,), (0,)), ((), ())),
                           preferred_element_type=F32, precision=jax.lax.Precision.HIGHEST)
    sub = lax.broadcasted_iota(jnp.int32, (Sb * K, Sb), 0)
    selm = (sub // K == lax.broadcasted_iota(
        jnp.int32, (Sb * K, Sb), 1)).astype(F32)
    y = jax.nn.relu(gx - jnp.dot(selm, corr, preferred_element_type=F32, precision=jax.lax.Precision.HIGHEST))
    y = jax.nn.relu(jnp.dot(y, w2_ref[...],
                            preferred_element_type=F32, precision=jax.lax.Precision.HIGHEST) + b2_ref[...])
    y = jax.nn.relu(jnp.dot(y, w3_ref[...],
                            preferred_element_type=F32, precision=jax.lax.Precision.HIGHEST) + b3_ref[...])
    cnt = jnp.maximum(cnt_ref[0][:, 0:1], 1.0)
    cexp = jnp.dot(selm, cnt, preferred_element_type=F32, precision=jax.lax.Precision.HIGHEST)
    kw = (lax.broadcasted_iota(jnp.int32, (Sb * K, 1), 0) % K).astype(F32)
    y = jnp.where(kw < cexp, y, NEG)
    o_ref[0] = jnp.max(y.reshape(Sb, K, c3), axis=1)


def _mlppool(gath, centrows, cnt, wx, w2, b2, w3, b3, K, Sb):
    c1 = gath.shape[1]
    B, S, _ = centrows.shape
    c3 = w3.shape[1]
    g4 = gath.reshape(B, S // Sb, Sb * K, c1)
    return pl.pallas_call(
        functools.partial(_mlppool_body, Sb=Sb, K=K, c3=c3),
        grid=(B, S // Sb),
        in_specs=[
            pl.BlockSpec((1, 1, Sb * K, c1), lambda b, j: (b, j, 0, 0)),
            pl.BlockSpec((1, Sb, 8), lambda b, j: (b, j, 0)),
            pl.BlockSpec((1, Sb, 8), lambda b, j: (b, j, 0)),
            pl.BlockSpec((8, c1), lambda b, j: (0, 0)),
            pl.BlockSpec((c1, w2.shape[1]), lambda b, j: (0, 0)),
            pl.BlockSpec((1, w2.shape[1]), lambda b, j: (0, 0)),
            pl.BlockSpec((w2.shape[1], c3), lambda b, j: (0, 0)),
            pl.BlockSpec((1, c3), lambda b, j: (0, 0)),
        ],
        out_specs=pl.BlockSpec((1, Sb, c3), lambda b, j: (b, j, 0)),
        out_shape=jax.ShapeDtypeStruct((B, S, c3), F32),
    )(g4, centrows, cnt, wx, w2, b2, w3, b3)


def _sa3_body(x_ref, w1_ref, b1_ref, w2_ref, b2_ref, w3_ref, b3_ref, o_ref):
    y = jax.nn.relu(jnp.dot(x_ref[0], w1_ref[...],
                            preferred_element_type=F32, precision=jax.lax.Precision.HIGHEST) + b1_ref[...])
    y = jax.nn.relu(jnp.dot(y, w2_ref[...],
                            preferred_element_type=F32, precision=jax.lax.Precision.HIGHEST) + b2_ref[...])
    y = jax.nn.relu(jnp.dot(y, w3_ref[...],
                            preferred_element_type=F32, precision=jax.lax.Precision.HIGHEST) + b3_ref[...])
    o_ref[0] = jnp.max(y, axis=0, keepdims=True)


def _sa3(x, w1, b1, w2, b2, w3, b3):
    B, N, Ci = x.shape
    c1, c2, c3 = w1.shape[1], w2.shape[1], w3.shape[1]
    return pl.pallas_call(
        _sa3_body,
        grid=(B,),
        in_specs=[
            pl.BlockSpec((1, N, Ci), lambda b: (b, 0, 0)),
            pl.BlockSpec((Ci, c1), lambda b: (0, 0)),
            pl.BlockSpec((1, c1), lambda b: (0, 0)),
            pl.BlockSpec((c1, c2), lambda b: (0, 0)),
            pl.BlockSpec((1, c2), lambda b: (0, 0)),
            pl.BlockSpec((c2, c3), lambda b: (0, 0)),
            pl.BlockSpec((1, c3), lambda b: (0, 0)),
        ],
        out_specs=pl.BlockSpec((1, 1, c3), lambda b: (b, 0, 0)),
        out_shape=jax.ShapeDtypeStruct((B, 1, c3), F32),
    )(x, w1, b1, w2, b2, w3, b3)


def _fp3_body(p1_ref, l3_ref, w1_ref, b1_ref, w2_ref, b2_ref, o_ref, *, N):
    l3 = jnp.broadcast_to(l3_ref[0], (N, l3_ref.shape[2]))
    x = jnp.concatenate([p1_ref[0], l3], axis=1)
    y = jax.nn.relu(jnp.dot(x, w1_ref[...],
                            preferred_element_type=F32, precision=jax.lax.Precision.HIGHEST) + b1_ref[...])
    y = jax.nn.relu(jnp.dot(y, w2_ref[...],
                            preferred_element_type=F32, precision=jax.lax.Precision.HIGHEST) + b2_ref[...])
    o_ref[0] = y


def _fp3(p1, l3, w1, b1, w2, b2):
    B, N, C1 = p1.shape
    c1, c2 = w1.shape[1], w2.shape[1]
    return pl.pallas_call(
        functools.partial(_fp3_body, N=N),
        grid=(B,),
        in_specs=[
            pl.BlockSpec((1, N, C1), lambda b: (b, 0, 0)),
            pl.BlockSpec((1, 1, l3.shape[2]), lambda b: (b, 0, 0)),
            pl.BlockSpec((w1.shape[0], c1), lambda b: (0, 0)),
            pl.BlockSpec((1, c1), lambda b: (0, 0)),
            pl.BlockSpec((c1, c2), lambda b: (0, 0)),
            pl.BlockSpec((1, c2), lambda b: (0, 0)),
        ],
        out_specs=pl.BlockSpec((1, N, c2), lambda b: (b, 0, 0)),
        out_shape=jax.ShapeDtypeStruct((B, N, c2), F32),
    )(p1, l3, w1, b1, w2, b2)


def _nn3_weights(D, S2):
    lane = lax.broadcasted_iota(jnp.int32, D.shape, 1)
    Wm = jnp.zeros(D.shape, F32)
    wsum = jnp.zeros((D.shape[0], 1), F32)
    for _ in range(3):
        dm = jnp.min(D, axis=1, keepdims=True)
        im = jnp.min(jnp.where(D == dm, lane, S2), axis=1, keepdims=True)
        wj = 1.0 / (dm + 1e-8)
        Wm = Wm + jnp.where(lane == im, wj, 0.0)
        wsum = wsum + wj
        D = jnp.where(lane == im, jnp.float32(1e30), D)
    return Wm / wsum


def _sqdist(a_rows, b_cols):
    G = lax.dot_general(a_rows.astype(jnp.bfloat16),
                        b_cols.astype(jnp.bfloat16), (((1,), (0,)), ((), ())),
                        preferred_element_type=F32)
    sn = jnp.sum(a_rows * a_rows, axis=1, keepdims=True)
    sx = jnp.sum(b_cols * b_cols, axis=0, keepdims=True)
    return sn + sx - 2.0 * G


def _fp2_body(x1_ref, x2_ref, p2_ref, p1_ref, w1_ref, b1_ref, w2_ref, b2_ref,
              o_ref, *, S2):
    D = _sqdist(x1_ref[0], x2_ref[0])
    Wm = _nn3_weights(D, S2)
    interp = jnp.dot(Wm, p2_ref[0], preferred_element_type=F32, precision=jax.lax.Precision.HIGHEST)
    x = jnp.concatenate([p1_ref[0], interp], axis=1)
    y = jax.nn.relu(jnp.dot(x, w1_ref[...],
                            preferred_element_type=F32, precision=jax.lax.Precision.HIGHEST) + b1_ref[...])
    y = jax.nn.relu(jnp.dot(y, w2_ref[...],
                            preferred_element_type=F32, precision=jax.lax.Precision.HIGHEST) + b2_ref[...])
    o_ref[0] = y


def _fp2(cent1rows, cent2, p2, p1, w1, b1, w2, b2):
    B, S1, _ = cent1rows.shape
    S2 = cent2.shape[2]
    C2 = p2.shape[2]
    C1 = p1.shape[2]
    c1, c2 = w1.shape[1], w2.shape[1]
    return pl.pallas_call(
        functools.partial(_fp2_body, S2=S2),
        grid=(B,),
        in_specs=[
            pl.BlockSpec((1, S1, 8), lambda b: (b, 0, 0)),
            pl.BlockSpec((1, 8, S2), lambda b: (b, 0, 0)),
            pl.BlockSpec((1, S2, C2), lambda b: (b, 0, 0)),
            pl.BlockSpec((1, S1, C1), lambda b: (b, 0, 0)),
            pl.BlockSpec((w1.shape[0], c1), lambda b: (0, 0)),
            pl.BlockSpec((1, c1), lambda b: (0, 0)),
            pl.BlockSpec((c1, c2), lambda b: (0, 0)),
            pl.BlockSpec((1, c2), lambda b: (0, 0)),
        ],
        out_specs=pl.BlockSpec((1, S1, c2), lambda b: (b, 0, 0)),
        out_shape=jax.ShapeDtypeStruct((B, S1, c2), F32),
    )(cent1rows, cent2, p2, p1, w1, b1, w2, b2)


def _fp1_body(x1_ref, x2_ref, p2_ref, w1_ref, b1_ref, w2_ref, b2_ref,
              wc1_ref, bc1_ref, wc2_ref, bc2_ref, o_ref, *, S2):
    x1r = x1_ref[0]
    D = _sqdist(x1r, x2_ref[0])
    Wm = _nn3_weights(D, S2)
    interp = jnp.dot(Wm, p2_ref[0], preferred_element_type=F32, precision=jax.lax.Precision.HIGHEST)
    x = jnp.concatenate([x1r[:, 0:3], x1r[:, 0:3], interp], axis=1)
    y = jax.nn.relu(jnp.dot(x, w1_ref[...],
                            preferred_element_type=F32, precision=jax.lax.Precision.HIGHEST) + b1_ref[...])
    y = jax.nn.relu(jnp.dot(y, w2_ref[...],
                            preferred_element_type=F32, precision=jax.lax.Precision.HIGHEST) + b2_ref[...])
    y = jax.nn.relu(jnp.dot(y, wc1_ref[...],
                            preferred_element_type=F32, precision=jax.lax.Precision.HIGHEST) + bc1_ref[...])
    lg = jnp.dot(y, wc2_ref[...], preferred_element_type=F32, precision=jax.lax.Precision.HIGHEST) + bc2_ref[...]
    m = jnp.max(lg, axis=1, keepdims=True)
    lse = jnp.log(jnp.sum(jnp.exp(lg - m), axis=1, keepdims=True)) + m
    o_ref[0] = lg - lse


def _fp1_head(x1rows, cent1, p2, w1, b1, w2, b2, wc1, bc1, wc2, bc2, Sr):
    B, N, _ = x1rows.shape
    S2 = cent1.shape[2]
    C2 = p2.shape[2]
    co = wc2.shape[1]
    return pl.pallas_call(
        functools.partial(_fp1_body, S2=S2),
        grid=(B, N // Sr),
        in_specs=[
            pl.BlockSpec((1, Sr, 8), lambda b, j: (b, j, 0)),
            pl.BlockSpec((1, 8, S2), lambda b, j: (b, 0, 0)),
            pl.BlockSpec((1, S2, C2), lambda b, j: (b, 0, 0)),
            pl.BlockSpec((w1.shape[0], w1.shape[1]), lambda b, j: (0, 0)),
            pl.BlockSpec((1, w1.shape[1]), lambda b, j: (0, 0)),
            pl.BlockSpec((w2.shape[0], w2.shape[1]), lambda b, j: (0, 0)),
            pl.BlockSpec((1, w2.shape[1]), lambda b, j: (0, 0)),
            pl.BlockSpec((wc1.shape[0], wc1.shape[1]), lambda b, j: (0, 0)),
            pl.BlockSpec((1, wc1.shape[1]), lambda b, j: (0, 0)),
            pl.BlockSpec((wc2.shape[0], co), lambda b, j: (0, 0)),
            pl.BlockSpec((1, co), lambda b, j: (0, 0)),
        ],
        out_specs=pl.BlockSpec((1, Sr, co), lambda b, j: (b, j, 0)),
        out_shape=jax.ShapeDtypeStruct((B, N, co), F32),
    )(x1rows, cent1, p2, w1, b1, w2, b2, wc1, bc1, wc2, bc2)


def kernel(xyz, params):
    B, _, N = xyz.shape
    S1, S2 = 512, 128

    xyzp = jnp.pad(xyz, ((0, 0), (0, 5), (0, 0)))
    x1rows = jnp.transpose(xyzp, (0, 2, 1))

    cent1 = _fps(xyzp, S1)
    cent1rows = jnp.transpose(cent1, (0, 2, 1))
    feat6 = jnp.concatenate([x1rows[:, :, 0:3], x1rows[:, :, 0:3],
                             jnp.zeros((B, N, 2), F32)], axis=2)

    sa1 = [(0.1, 32, 'sa1_b0'), (0.2, 64, 'sa1_b1'), (0.4, 128, 'sa1_b2')]
    outs1 = []
    for radius, K, name in sa1:
        (w1t, b1), (w2t, b2), (w3t, b3) = [_fold(l) for l in params[name]]
        c1 = w1t.shape[1]
        w1p = jnp.zeros((8, 128), F32)
        w1p = w1p.at[0:3, 0:c1].set(w1t[0:3] + w1t[3:6])
        wx = jnp.zeros((8, 128), F32).at[0:3, 0:c1].set(w1t[3:6])
        b1p = jnp.zeros((1, 128), F32).at[:, 0:c1].set(b1)
        w2p = jnp.zeros((128, w2t.shape[1]), F32).at[0:c1].set(w2t)
        U = _dense(feat6, w1p, b1p)
        gidx, cnt = _ballq(xyzp, cent1rows, K, radius * radius, 128)
        rows = _sc_gather(U.reshape(B * N, 128), gidx.reshape(-1))
        outs1.append(_mlppool(rows, cent1rows, cnt, wx, w2p, b2, w3t, b3,
                              K, 64))
    l1_points = jnp.concatenate(outs1, axis=2)

    cent2 = _fps(cent1, S2)
    cent2rows = jnp.transpose(cent2, (0, 2, 1))
    feat2 = jnp.concatenate([l1_points, cent1rows[:, :, 0:3],
                             jnp.zeros((B, S1, 13), F32)], axis=2)

    sa2 = [(0.4, 64, 'sa2_b0'), (0.8, 128, 'sa2_b1')]
    outs2 = []
    for radius, K, name in sa2:
        (w1t, b1), (w2t, b2), (w3t, b3) = [_fold(l) for l in params[name]]
        c1 = w1t.shape[1]
        w1p = jnp.zeros((336, c1), F32).at[0:323].set(w1t)
        wx = jnp.zeros((8, c1), F32).at[0:3].set(w1t[320:323])
        U = _dense(feat2, w1p, b1)
        gidx, cnt = _ballq(cent1, cent2rows, K, radius * radius, 128)
        rows = _sc_gather(U.reshape(B * S1, c1), gidx.reshape(-1))
        outs2.append(_mlppool(rows, cent2rows, cnt, wx, w2t, b2, w3t, b3,
                              K, 16))
    l2_points = jnp.concatenate(outs2, axis=2)

    (w1t, b1), (w2t, b2), (w3t, b3) = [_fold(l) for l in params['sa3']]
    w1p = jnp.zeros((520, w1t.shape[1]), F32).at[0:515].set(w1t)
    feat3 = jnp.concatenate([cent2rows[:, :, 0:3], l2_points,
                             jnp.zeros((B, S2, 5), F32)], axis=2)
    l3 = _sa3(feat3, w1p, b1, w2t, b2, w3t, b3)

    (w1t, b1), (w2t, b2) = [_fold(l) for l in params['fp3']]
    p2 = _fp3(l2_points, l3, w1t, b1, w2t, b2)

    (w1t, b1), (w2t, b2) = [_fold(l) for l in params['fp2']]
    p1 = _fp2(cent1rows, cent2, p2, l1_points, w1t, b1, w2t, b2)

    (w1t, b1), (w2t, b2) = [_fold(l) for l in params['fp1']]
    (wc1, bc1) = _fold(params['conv1'])
    wc2 = params['conv2']['w'].T
    wc2p = jnp.zeros((128, 16), F32).at[:, 0:13].set(wc2)
    bc2p = jnp.full((1, 16), NEG, F32).at[0, 0:13].set(params['conv2']['b'])
    x = _fp1_head(x1rows, cent1, p1, w1t, b1, w2t, b2,
                  wc1, bc1, wc2p, bc2p, 1024)

    return x[:, :, 0:13], jnp.transpose(l3, (0, 2, 1))


# speedup vs baseline: 5.7214x; 1.1356x over previous
import functools

import jax
import jax.numpy as jnp
from jax import lax
from jax.experimental import pallas as pl
from jax.experimental.pallas import tpu as pltpu
from jax.experimental.pallas import tpu_sc as plsc

F32 = jnp.float32
NEG = -1e30


def _fold(layer):
    s = layer['gamma'] / jnp.sqrt(1.0 + 1e-5)
    w = layer['w'] * s[:, None]
    b = layer['b'] * s + layer['beta']
    return w.T, b[None, :]


def _fps_body(x_ref, o_ref, *, N, S):
    x = x_ref[0, 0:1]
    y = x_ref[0, 1:2]
    z = x_ref[0, 2:3]
    lane = lax.broadcasted_iota(jnp.int32, (1, N), 1)
    laneS = lax.broadcasted_iota(jnp.int32, (1, S), 1)

    def step(i, carry):
        dist, far, cx, cy, cz = carry
        sel = (lane == far).astype(F32)
        xf = jnp.sum(x * sel, axis=1, keepdims=True)
        yf = jnp.sum(y * sel, axis=1, keepdims=True)
        zf = jnp.sum(z * sel, axis=1, keepdims=True)
        at_i = laneS == i
        cx = jnp.where(at_i, xf, cx)
        cy = jnp.where(at_i, yf, cy)
        cz = jnp.where(at_i, zf, cz)
        d = (x - xf) ** 2 + (y - yf) ** 2 + (z - zf) ** 2
        dist = jnp.minimum(dist, d)
        m = jnp.max(dist)
        far = jnp.min(jnp.where(dist == m, lane, N)).astype(jnp.int32)
        return dist, far, cx, cy, cz

    zS = jnp.zeros((1, S), F32)
    init = (jnp.full((1, N), 1e10, F32), jnp.int32(0), zS, zS, zS)
    _, _, cx, cy, cz = lax.fori_loop(0, S, step, init)
    o_ref[0] = jnp.concatenate([cx, cy, cz, jnp.zeros((5, S), F32)], axis=0)


def _fps(xyzp, S):
    B, _, N = xyzp.shape
    return pl.pallas_call(
        functools.partial(_fps_body, N=N, S=S),
        grid=(B,),
        in_specs=[pl.BlockSpec((1, 8, N), lambda b: (b, 0, 0))],
        out_specs=pl.BlockSpec((1, 8, S), lambda b: (b, 0, 0)),
        out_shape=jax.ShapeDtypeStruct((B, 8, S), F32),
    )(xyzp)


def _probe(rank, pos, nspans):
    hi = pos // 128
    lo = pos - hi * 128
    acc = jnp.zeros(pos.shape, F32)
    for j in range(nspans):
        g = jnp.take_along_axis(rank[:, j * 128:(j + 1) * 128], lo, axis=1)
        acc = jnp.where(hi == j, g, acc)
    return acc


def _ballq_body(x_ref, c_ref, gi_ref, cnt_ref, *, N, K, r2, Sq):
    xp = x_ref[0]
    ce = c_ref[0]
    G = lax.dot_general(ce.astype(jnp.bfloat16), xp.astype(jnp.bfloat16),
                        (((1,), (0,)), ((), ())),
                        preferred_element_type=F32)
    sn = jnp.sum(ce * ce, axis=1, keepdims=True)
    sx = jnp.sum(xp * xp, axis=0, keepdims=True)
    D = sn + sx - 2.0 * G
    mask = (D <= r2).astype(F32)

    lane = lax.broadcasted_iota(jnp.int32, (Sq, N), 1)
    r = mask
    sh = 1
    while sh < N:
        r = r + jnp.where(lane >= sh, jnp.roll(r, sh, axis=1), 0.0)
        sh *= 2
    cnt = r[:, N - 1:N]
    cnt_ref[0] = jnp.broadcast_to(cnt, (Sq, 8))

    kk = (lax.broadcasted_iota(jnp.int32, (Sq, K), 1) + 1).astype(F32)
    lo = jnp.full((Sq, K), -1, jnp.int32)
    b = 1
    while b < N:
        b *= 2
    while b >= 1:
        nxt = lo + b
        v = _probe(r, jnp.minimum(nxt, N - 1), N // 128)
        ok = (nxt <= N - 1) & (v < kk)
        lo = jnp.where(ok, nxt, lo)
        b //= 2
    g = jnp.minimum(lo + 1, N - 1)
    gi_ref[0] = g + pl.program_id(0) * N


def _ballq(xyzp, centrows, K, r2, Sq):
    B, _, N = xyzp.shape
    S = centrows.shape[1]
    return pl.pallas_call(
        functools.partial(_ballq_body, N=N, K=K, r2=r2, Sq=Sq),
        grid=(B, S // Sq),
        in_specs=[
            pl.BlockSpec((1, 8, N), lambda b, j: (b, 0, 0)),
            pl.BlockSpec((1, Sq, 8), lambda b, j: (b, j, 0)),
        ],
        out_specs=[
            pl.BlockSpec((1, Sq, K), lambda b, j: (b, j, 0)),
            pl.BlockSpec((1, Sq, 8), lambda b, j: (b, j, 0)),
        ],
        out_shape=[
            jax.ShapeDtypeStruct((B, S, K), jnp.int32),
            jax.ShapeDtypeStruct((B, S, 8), F32),
        ],
    )(xyzp, centrows)


def _sc_gather(table, idx):
    R, Dc = table.shape
    (Bt,) = idx.shape
    info = plsc.get_sparse_core_info()
    NW = info.num_cores * info.num_subcores
    b_per_w = Bt // NW
    limit = 120000 // (Dc + 1)
    chunk = b_per_w
    while chunk > limit:
        chunk //= 2
    nchunks = b_per_w // chunk
    mesh = plsc.VectorSubcoreMesh(core_axis_name="c", subcore_axis_name="s")

    @functools.partial(
        pl.kernel, mesh=mesh,
        out_type=jax.ShapeDtypeStruct((Bt, Dc), F32),
        scratch_types=[
            pltpu.VMEM((chunk,), jnp.int32),
            pltpu.VMEM((chunk, Dc), F32),
            pltpu.SemaphoreType.DMA,
        ],
    )
    def k(table_hbm, idx_hbm, out_hbm, idx_v, rows_v, sem):
        wid = lax.axis_index("s") * info.num_cores + lax.axis_index("c")
        base = wid * b_per_w

        def body(i, c):
            off = base + i * chunk
            pltpu.sync_copy(idx_hbm.at[pl.ds(off, chunk)], idx_v)
            pltpu.async_copy(table_hbm.at[idx_v], rows_v, sem).wait()
            pltpu.sync_copy(rows_v, out_hbm.at[pl.ds(off, chunk)])
            return c

        lax.fori_loop(0, nchunks, body, 0)

    return k(table, idx)


def _dense_body(x_ref, w_ref, b_ref, o_ref):
    o_ref[0] = jnp.dot(x_ref[0], w_ref[...],
                       preferred_element_type=F32) + b_ref[...]


def _dense(x, w, b):
    B, N, Ci = x.shape
    Co = w.shape[1]
    return pl.pallas_call(
        _dense_body,
        grid=(B,),
        in_specs=[
            pl.BlockSpec((1, N, Ci), lambda bb: (bb, 0, 0)),
            pl.BlockSpec((Ci, Co), lambda bb: (0, 0)),
            pl.BlockSpec((1, Co), lambda bb: (0, 0)),
        ],
        out_specs=pl.BlockSpec((1, N, Co), lambda bb: (bb, 0, 0)),
        out_shape=jax.ShapeDtypeStruct((B, N, Co), F32),
    )(x, w, b)


def _mlppool_body(g_ref, c_ref, cnt_ref, wx_ref, w2_ref, b2_ref, w3_ref,
                  b3_ref, o_ref, *, Sb, K, c3):
    gx = g_ref[0, 0]
    ce = c_ref[0]
    corr = lax.dot_general(ce, wx_ref[...], (((1,), (0,)), ((), ())),
                           preferred_element_type=F32,
                           precision=jax.lax.Precision.HIGHEST)
    sub = lax.broadcasted_iota(jnp.int32, (Sb * K, Sb), 0)
    selm = (sub // K == lax.broadcasted_iota(
        jnp.int32, (Sb * K, Sb), 1)).astype(F32)
    y = jax.nn.relu(gx - jnp.dot(selm, corr, preferred_element_type=F32, precision=jax.lax.Precision.HIGHEST))
    y = jax.nn.relu(jnp.dot(y, w2_ref[...],
                            preferred_element_type=F32) + b2_ref[...])
    y = jax.nn.relu(jnp.dot(y, w3_ref[...],
                            preferred_element_type=F32) + b3_ref[...])
    cnt = jnp.maximum(cnt_ref[0][:, 0:1], 1.0)
    cexp = jnp.dot(selm, cnt, preferred_element_type=F32, precision=jax.lax.Precision.HIGHEST)
    kw = (lax.broadcasted_iota(jnp.int32, (Sb * K, 1), 0) % K).astype(F32)
    y = jnp.where(kw < cexp, y, NEG)
    o_ref[0] = jnp.max(y.reshape(Sb, K, c3), axis=1)


def _mlppool(gath, centrows, cnt, wx, w2, b2, w3, b3, K, Sb):
    c1 = gath.shape[1]
    B, S, _ = centrows.shape
    c3 = w3.shape[1]
    g4 = gath.reshape(B, S // Sb, Sb * K, c1)
    return pl.pallas_call(
        functools.partial(_mlppool_body, Sb=Sb, K=K, c3=c3),
        grid=(B, S // Sb),
        in_specs=[
            pl.BlockSpec((1, 1, Sb * K, c1), lambda b, j: (b, j, 0, 0)),
            pl.BlockSpec((1, Sb, 8), lambda b, j: (b, j, 0)),
            pl.BlockSpec((1, Sb, 8), lambda b, j: (b, j, 0)),
            pl.BlockSpec((8, c1), lambda b, j: (0, 0)),
            pl.BlockSpec((c1, w2.shape[1]), lambda b, j: (0, 0)),
            pl.BlockSpec((1, w2.shape[1]), lambda b, j: (0, 0)),
            pl.BlockSpec((w2.shape[1], c3), lambda b, j: (0, 0)),
            pl.BlockSpec((1, c3), lambda b, j: (0, 0)),
        ],
        out_specs=pl.BlockSpec((1, Sb, c3), lambda b, j: (b, j, 0)),
        out_shape=jax.ShapeDtypeStruct((B, S, c3), F32),
    )(g4, centrows, cnt, wx, w2, b2, w3, b3)


def _sa3_body(x_ref, w1_ref, b1_ref, w2_ref, b2_ref, w3_ref, b3_ref, o_ref):
    y = jax.nn.relu(jnp.dot(x_ref[0], w1_ref[...],
                            preferred_element_type=F32) + b1_ref[...])
    y = jax.nn.relu(jnp.dot(y, w2_ref[...],
                            preferred_element_type=F32) + b2_ref[...])
    y = jax.nn.relu(jnp.dot(y, w3_ref[...],
                            preferred_element_type=F32) + b3_ref[...])
    o_ref[0] = jnp.max(y, axis=0, keepdims=True)


def _sa3(x, w1, b1, w2, b2, w3, b3):
    B, N, Ci = x.shape
    c1, c2, c3 = w1.shape[1], w2.shape[1], w3.shape[1]
    return pl.pallas_call(
        _sa3_body,
        grid=(B,),
        in_specs=[
            pl.BlockSpec((1, N, Ci), lambda b: (b, 0, 0)),
            pl.BlockSpec((Ci, c1), lambda b: (0, 0)),
            pl.BlockSpec((1, c1), lambda b: (0, 0)),
            pl.BlockSpec((c1, c2), lambda b: (0, 0)),
            pl.BlockSpec((1, c2), lambda b: (0, 0)),
            pl.BlockSpec((c2, c3), lambda b: (0, 0)),
            pl.BlockSpec((1, c3), lambda b: (0, 0)),
        ],
        out_specs=pl.BlockSpec((1, 1, c3), lambda b: (b, 0, 0)),
        out_shape=jax.ShapeDtypeStruct((B, 1, c3), F32),
    )(x, w1, b1, w2, b2, w3, b3)


def _fp3_body(p1_ref, l3_ref, w1_ref, b1_ref, w2_ref, b2_ref, o_ref, *, N):
    l3 = jnp.broadcast_to(l3_ref[0], (N, l3_ref.shape[2]))
    x = jnp.concatenate([p1_ref[0], l3], axis=1)
    y = jax.nn.relu(jnp.dot(x, w1_ref[...],
                            preferred_element_type=F32) + b1_ref[...])
    y = jax.nn.relu(jnp.dot(y, w2_ref[...],
                            preferred_element_type=F32) + b2_ref[...])
    o_ref[0] = y


def _fp3(p1, l3, w1, b1, w2, b2):
    B, N, C1 = p1.shape
    c1, c2 = w1.shape[1], w2.shape[1]
    return pl.pallas_call(
        functools.partial(_fp3_body, N=N),
        grid=(B,),
        in_specs=[
            pl.BlockSpec((1, N, C1), lambda b: (b, 0, 0)),
            pl.BlockSpec((1, 1, l3.shape[2]), lambda b: (b, 0, 0)),
            pl.BlockSpec((w1.shape[0], c1), lambda b: (0, 0)),
            pl.BlockSpec((1, c1), lambda b: (0, 0)),
            pl.BlockSpec((c1, c2), lambda b: (0, 0)),
            pl.BlockSpec((1, c2), lambda b: (0, 0)),
        ],
        out_specs=pl.BlockSpec((1, N, c2), lambda b: (b, 0, 0)),
        out_shape=jax.ShapeDtypeStruct((B, N, c2), F32),
    )(p1, l3, w1, b1, w2, b2)


def _nn3_weights(D, S2):
    lane = lax.broadcasted_iota(jnp.int32, D.shape, 1)
    Wm = jnp.zeros(D.shape, F32)
    wsum = jnp.zeros((D.shape[0], 1), F32)
    for _ in range(3):
        dm = jnp.min(D, axis=1, keepdims=True)
        im = jnp.min(jnp.where(D == dm, lane, S2), axis=1, keepdims=True)
        wj = 1.0 / (dm + 1e-8)
        Wm = Wm + jnp.where(lane == im, wj, 0.0)
        wsum = wsum + wj
        D = jnp.where(lane == im, jnp.float32(1e30), D)
    return Wm / wsum


def _sqdist(a_rows, b_cols):
    G = lax.dot_general(a_rows.astype(jnp.bfloat16),
                        b_cols.astype(jnp.bfloat16), (((1,), (0,)), ((), ())),
                        preferred_element_type=F32)
    sn = jnp.sum(a_rows * a_rows, axis=1, keepdims=True)
    sx = jnp.sum(b_cols * b_cols, axis=0, keepdims=True)
    return sn + sx - 2.0 * G


def _fp2_body(x1_ref, x2_ref, p2_ref, p1_ref, w1_ref, b1_ref, w2_ref, b2_ref,
              o_ref, *, S2):
    D = _sqdist(x1_ref[0], x2_ref[0])
    Wm = _nn3_weights(D, S2)
    interp = jnp.dot(Wm, p2_ref[0], preferred_element_type=F32)
    x = jnp.concatenate([p1_ref[0], interp], axis=1)
    y = jax.nn.relu(jnp.dot(x, w1_ref[...],
                            preferred_element_type=F32) + b1_ref[...])
    y = jax.nn.relu(jnp.dot(y, w2_ref[...],
                            preferred_element_type=F32) + b2_ref[...])
    o_ref[0] = y


def _fp2(cent1rows, cent2, p2, p1, w1, b1, w2, b2):
    B, S1, _ = cent1rows.shape
    S2 = cent2.shape[2]
    C2 = p2.shape[2]
    C1 = p1.shape[2]
    c1, c2 = w1.shape[1], w2.shape[1]
    return pl.pallas_call(
        functools.partial(_fp2_body, S2=S2),
        grid=(B,),
        in_specs=[
            pl.BlockSpec((1, S1, 8), lambda b: (b, 0, 0)),
            pl.BlockSpec((1, 8, S2), lambda b: (b, 0, 0)),
            pl.BlockSpec((1, S2, C2), lambda b: (b, 0, 0)),
            pl.BlockSpec((1, S1, C1), lambda b: (b, 0, 0)),
            pl.BlockSpec((w1.shape[0], c1), lambda b: (0, 0)),
            pl.BlockSpec((1, c1), lambda b: (0, 0)),
            pl.BlockSpec((c1, c2), lambda b: (0, 0)),
            pl.BlockSpec((1, c2), lambda b: (0, 0)),
        ],
        out_specs=pl.BlockSpec((1, S1, c2), lambda b: (b, 0, 0)),
        out_shape=jax.ShapeDtypeStruct((B, S1, c2), F32),
    )(cent1rows, cent2, p2, p1, w1, b1, w2, b2)


def _fp1_body(x1_ref, x2_ref, p2_ref, w1_ref, b1_ref, w2_ref, b2_ref,
              wc1_ref, bc1_ref, wc2_ref, bc2_ref, o_ref, *, S2):
    x1r = x1_ref[0]
    D = _sqdist(x1r, x2_ref[0])
    Wm = _nn3_weights(D, S2)
    interp = jnp.dot(Wm, p2_ref[0], preferred_element_type=F32)
    x = jnp.concatenate([x1r[:, 0:3], x1r[:, 0:3], interp], axis=1)
    y = jax.nn.relu(jnp.dot(x, w1_ref[...],
                            preferred_element_type=F32) + b1_ref[...])
    y = jax.nn.relu(jnp.dot(y, w2_ref[...],
                            preferred_element_type=F32) + b2_ref[...])
    y = jax.nn.relu(jnp.dot(y, wc1_ref[...],
                            preferred_element_type=F32) + bc1_ref[...])
    lg = jnp.dot(y, wc2_ref[...], preferred_element_type=F32) + bc2_ref[...]
    m = jnp.max(lg, axis=1, keepdims=True)
    lse = jnp.log(jnp.sum(jnp.exp(lg - m), axis=1, keepdims=True)) + m
    o_ref[0] = lg - lse


def _fp1_head(x1rows, cent1, p2, w1, b1, w2, b2, wc1, bc1, wc2, bc2, Sr):
    B, N, _ = x1rows.shape
    S2 = cent1.shape[2]
    C2 = p2.shape[2]
    co = wc2.shape[1]
    return pl.pallas_call(
        functools.partial(_fp1_body, S2=S2),
        grid=(B, N // Sr),
        in_specs=[
            pl.BlockSpec((1, Sr, 8), lambda b, j: (b, j, 0)),
            pl.BlockSpec((1, 8, S2), lambda b, j: (b, 0, 0)),
            pl.BlockSpec((1, S2, C2), lambda b, j: (b, 0, 0)),
            pl.BlockSpec((w1.shape[0], w1.shape[1]), lambda b, j: (0, 0)),
            pl.BlockSpec((1, w1.shape[1]), lambda b, j: (0, 0)),
            pl.BlockSpec((w2.shape[0], w2.shape[1]), lambda b, j: (0, 0)),
            pl.BlockSpec((1, w2.shape[1]), lambda b, j: (0, 0)),
            pl.BlockSpec((wc1.shape[0], wc1.shape[1]), lambda b, j: (0, 0)),
            pl.BlockSpec((1, wc1.shape[1]), lambda b, j: (0, 0)),
            pl.BlockSpec((wc2.shape[0], co), lambda b, j: (0, 0)),
            pl.BlockSpec((1, co), lambda b, j: (0, 0)),
        ],
        out_specs=pl.BlockSpec((1, Sr, co), lambda b, j: (b, j, 0)),
        out_shape=jax.ShapeDtypeStruct((B, N, co), F32),
    )(x1rows, cent1, p2, w1, b1, w2, b2, wc1, bc1, wc2, bc2)


def kernel(xyz, params):
    B, _, N = xyz.shape
    S1, S2 = 512, 128

    xyzp = jnp.pad(xyz, ((0, 0), (0, 5), (0, 0)))
    x1rows = jnp.transpose(xyzp, (0, 2, 1))

    cent1 = _fps(xyzp, S1)
    cent1rows = jnp.transpose(cent1, (0, 2, 1))
    feat6 = jnp.concatenate([x1rows[:, :, 0:3], x1rows[:, :, 0:3],
                             jnp.zeros((B, N, 2), F32)], axis=2)

    sa1 = [(0.1, 32, 'sa1_b0'), (0.2, 64, 'sa1_b1'), (0.4, 128, 'sa1_b2')]
    outs1 = []
    for radius, K, name in sa1:
        (w1t, b1), (w2t, b2), (w3t, b3) = [_fold(l) for l in params[name]]
        c1 = w1t.shape[1]
        w1p = jnp.zeros((8, 128), F32)
        w1p = w1p.at[0:3, 0:c1].set(w1t[0:3] + w1t[3:6])
        wx = jnp.zeros((8, 128), F32).at[0:3, 0:c1].set(w1t[3:6])
        b1p = jnp.zeros((1, 128), F32).at[:, 0:c1].set(b1)
        w2p = jnp.zeros((128, w2t.shape[1]), F32).at[0:c1].set(w2t)
        U = _dense(feat6, w1p, b1p)
        gidx, cnt = _ballq(xyzp, cent1rows, K, radius * radius, 128)
        rows = _sc_gather(U.reshape(B * N, 128), gidx.reshape(-1))
        outs1.append(_mlppool(rows, cent1rows, cnt, wx, w2p, b2, w3t, b3,
                              K, 64))
    l1_points = jnp.concatenate(outs1, axis=2)

    cent2 = _fps(cent1, S2)
    cent2rows = jnp.transpose(cent2, (0, 2, 1))
    feat2 = jnp.concatenate([l1_points, cent1rows[:, :, 0:3],
                             jnp.zeros((B, S1, 13), F32)], axis=2)

    sa2 = [(0.4, 64, 'sa2_b0'), (0.8, 128, 'sa2_b1')]
    outs2 = []
    for radius, K, name in sa2:
        (w1t, b1), (w2t, b2), (w3t, b3) = [_fold(l) for l in params[name]]
        c1 = w1t.shape[1]
        w1p = jnp.zeros((336, c1), F32).at[0:323].set(w1t)
        wx = jnp.zeros((8, c1), F32).at[0:3].set(w1t[320:323])
        U = _dense(feat2, w1p, b1)
        gidx, cnt = _ballq(cent1, cent2rows, K, radius * radius, 128)
        rows = _sc_gather(U.reshape(B * S1, c1), gidx.reshape(-1))
        outs2.append(_mlppool(rows, cent2rows, cnt, wx, w2t, b2, w3t, b3,
                              K, 16))
    l2_points = jnp.concatenate(outs2, axis=2)

    (w1t, b1), (w2t, b2), (w3t, b3) = [_fold(l) for l in params['sa3']]
    w1p = jnp.zeros((520, w1t.shape[1]), F32).at[0:515].set(w1t)
    feat3 = jnp.concatenate([cent2rows[:, :, 0:3], l2_points,
                             jnp.zeros((B, S2, 5), F32)], axis=2)
    l3 = _sa3(feat3, w1p, b1, w2t, b2, w3t, b3)

    (w1t, b1), (w2t, b2) = [_fold(l) for l in params['fp3']]
    p2 = _fp3(l2_points, l3, w1t, b1, w2t, b2)

    (w1t, b1), (w2t, b2) = [_fold(l) for l in params['fp2']]
    p1 = _fp2(cent1rows, cent2, p2, l1_points, w1t, b1, w2t, b2)

    (w1t, b1), (w2t, b2) = [_fold(l) for l in params['fp1']]
    (wc1, bc1) = _fold(params['conv1'])
    wc2 = params['conv2']['w'].T
    wc2p = jnp.zeros((128, 16), F32).at[:, 0:13].set(wc2)
    bc2p = jnp.full((1, 16), NEG, F32).at[0, 0:13].set(params['conv2']['b'])
    x = _fp1_head(x1rows, cent1, p1, w1t, b1, w2t, b2,
                  wc1, bc1, wc2p, bc2p, 1024)

    return x[:, :, 0:13], jnp.transpose(l3, (0, 2, 1))
